# baseline (speedup 1.0000x reference)
import functools

import jax
import jax.numpy as jnp
from jax import lax
from jax.experimental import pallas as pl
from jax.experimental.pallas import tpu as pltpu
from jax.experimental.pallas import tpu_sc as plsc

G, N, E, D, OUT, DE = 6, 1024, 32768, 128, 128, 16
GI = G * 4

_INTERPRET = False


def _gcn_body(x_ref, w_ref, b_ref, ws_ref, Bt_ref,
              gs_ref, xs_ref, acc_ref):
    i = pl.program_id(1)
    xw = jnp.dot(x_ref[0], w_ref[0], preferred_element_type=jnp.float32)
    deg = jnp.zeros((N, 1), jnp.float32)
    for tc in range(8):
        B2 = Bt_ref[0, :, tc].reshape(N, OUT)
        deg = deg + jnp.sum(B2, axis=1, keepdims=True)
    dis = jax.lax.rsqrt(deg + 1.0)
    z = dis * xw
    zb = z.astype(jnp.bfloat16)
    m = jnp.zeros((N, OUT), jnp.float32)
    for tc in range(8):
        B2 = Bt_ref[0, :, tc].reshape(N, OUT).astype(jnp.bfloat16)
        m = m + jnp.dot(B2, zb[tc * 128:(tc + 1) * 128, :],
                        preferred_element_type=jnp.float32)
    y = dis * m + (dis * dis) * xw + b_ref[0]
    xt = jnp.maximum(y, 0.0)
    contrib = jnp.dot(xt, ws_ref[0], preferred_element_type=jnp.float32)

    @pl.when(i == 0)
    def _():
        acc_ref[...] = contrib

    @pl.when(i != 0)
    def _():
        acc_ref[...] = acc_ref[...] + contrib

    @pl.when(i == 3)
    def _():
        t = acc_ref[...]
        gs_ref[0, 0] = jnp.mean(t, axis=0)
        nrm = jnp.sqrt(jnp.sum(t * t, axis=1, keepdims=True))
        xs_ref[0] = t / jnp.maximum(nrm, 1e-12)


def _run_gcn(x, W_gcn, b_gcn, Ws, Bt, g0, ng):
    b3 = b_gcn.reshape(4, 1, OUT)
    gs, xs = pl.pallas_call(
        _gcn_body,
        grid=(ng, 4),
        in_specs=[
            pl.BlockSpec((1, N, D), lambda g, i: (g0 + g, 0, 0)),
            pl.BlockSpec((1, D, OUT), lambda g, i: (i, 0, 0)),
            pl.BlockSpec((1, 1, OUT), lambda g, i: (i, 0, 0)),
            pl.BlockSpec((1, OUT, OUT), lambda g, i: (i, 0, 0)),
            pl.BlockSpec((1, 128, 8, 8, 128),
                         lambda g, i: (g * 4 + i, 0, 0, 0, 0)),
        ],
        out_specs=[
            pl.BlockSpec((1, 1, OUT), lambda g, i: (g, 0, 0)),
            pl.BlockSpec((1, N, OUT), lambda g, i: (g, 0, 0)),
        ],
        out_shape=[
            jax.ShapeDtypeStruct((ng, 1, OUT), jnp.float32),
            jax.ShapeDtypeStruct((ng, N, OUT), jnp.float32),
        ],
        scratch_shapes=[pltpu.VMEM((N, OUT), jnp.float32)],
        interpret=_INTERPRET,
    )(x, W_gcn, b3, Ws, Bt)
    return gs, xs


def _sim_body(a_ref, b_ref, out_ref):
    i = pl.program_id(0)
    j = pl.program_id(1)

    @pl.when(j >= i)
    def _():
        a16 = a_ref[0].astype(jnp.bfloat16)
        b16 = b_ref[0].astype(jnp.bfloat16)
        p = jax.lax.dot_general(a16, b16, (((1,), (1,)), ((), ())),
                                preferred_element_type=jnp.float32)
        t = jnp.tanh(p)
        f = jnp.floor((t + 1.0) * 10.5) - 10.0
        col = jnp.sum(f, axis=0)
        out_ref[0, 0] = jnp.sum(col.reshape(8, OUT), axis=0)


def _run_sim(xs):
    part = pl.pallas_call(
        _sim_body,
        grid=(G, G),
        in_specs=[
            pl.BlockSpec((1, N, OUT), lambda i, j: (i, 0, 0)),
            pl.BlockSpec((1, N, OUT), lambda i, j: (j, 0, 0)),
        ],
        out_specs=pl.BlockSpec((1, 1, OUT), lambda i, j: (i * G + j, 0, 0)),
        out_shape=jax.ShapeDtypeStruct((G * G, 1, OUT), jnp.float32),
        interpret=_INTERPRET,
    )(xs, xs)
    sums = jnp.sum(part.reshape(G * G, OUT), axis=-1)
    s = (0.1 / (N * N)) * sums.reshape(G, G)
    ut = jnp.triu(jnp.ones((G, G), jnp.bool_))
    return jnp.where(ut, s, s.T)


_NS = 16
_L = 16
_EPT = E // _NS
_CH = 128
_NCH = _EPT // _CH
_STRIPE = (N * N) // _NS
_ZB = 16384


def _sc_build_body(ei_hbm, ew_hbm, B_hbm,
                   B_sh, zeros_v, dstv, srcv, idx2d, val2d, zval2d,
                   *, j0, nj):
    c = lax.axis_index("c")
    s = lax.axis_index("s")

    def _zinit(k, _):
        zeros_v[pl.ds(k * _L, _L)] = jnp.zeros((_L,), jnp.float32)
        return _
    lax.fori_loop(0, _ZB // _L, _zinit, None)

    def _zinit2(k, _):
        zval2d[k // (_CH // _L), pl.ds((k % (_CH // _L)) * _L, _L)] = \
            jnp.zeros((_L,), jnp.float32)
        return _
    lax.fori_loop(0, _EPT // _L, _zinit2, None)

    for q in range(_STRIPE // _ZB):
        pltpu.sync_copy(zeros_v, B_sh.at[pl.ds(s * _STRIPE + q * _ZB, _ZB)])

    def _round(r, _):
        j = j0 + 2 * r + c
        g = j // 4
        i = j % 4

        pltpu.sync_copy(ei_hbm.at[g, 0, pl.ds(s * _EPT, _EPT)], srcv)
        pltpu.sync_copy(ei_hbm.at[g, 1, pl.ds(s * _EPT, _EPT)], dstv)
        pltpu.sync_copy(ew_hbm.at[g, i, s], val2d)

        for t in range(_NCH):
            def _grp(kk, _):
                k = t * (_CH // _L) + kk
                d16 = dstv[pl.ds(k * _L, _L)]
                s16 = srcv[pl.ds(k * _L, _L)]
                addr = ((d16 >> 3) << 13) | ((s16 >> 7) << 10) \
                    | ((d16 & 7) << 7) | (s16 & 127)
                idx2d[t, pl.ds(kk * _L, _L)] = addr
                return _
            lax.fori_loop(0, _CH // _L, _grp, None)

        plsc.subcore_barrier()

        for t in range(_NCH):
            pltpu.sync_copy(val2d.at[t], B_sh.at[idx2d.at[t]], add=True)
        plsc.subcore_barrier()

        pltpu.sync_copy(B_sh.at[pl.ds(s * _STRIPE, _STRIPE)],
                        B_hbm.at[pl.ds((j - j0) * (N * N) + s * _STRIPE,
                                       _STRIPE)])
        plsc.subcore_barrier()

        @pl.when(r < nj // 2 - 1)
        def _():
            for t in range(_NCH):
                pltpu.sync_copy(zval2d.at[t], B_sh.at[idx2d.at[t]])
        return _

    lax.fori_loop(0, nj // 2, _round, None)


def _build_B_sc(edge_index, ew, j0, nj):
    run = pl.kernel(
        functools.partial(_sc_build_body, j0=j0, nj=nj),
        mesh=plsc.VectorSubcoreMesh(core_axis_name="c", subcore_axis_name="s"),
        out_type=jax.ShapeDtypeStruct((nj * N * N,), jnp.float32),
        scratch_types=[
            pltpu.VMEM_SHARED((N * N,), jnp.float32),
            pltpu.VMEM((_ZB,), jnp.float32),
            pltpu.VMEM((_EPT,), jnp.int32),
            pltpu.VMEM((_EPT,), jnp.int32),
            pltpu.VMEM((_NCH, _CH), jnp.int32),
            pltpu.VMEM((_NCH, _CH), jnp.float32),
            pltpu.VMEM((_NCH, _CH), jnp.float32),
        ],
    )
    B = run(edge_index, ew)
    return B.reshape(nj, N // 8, 8, 8, 128)


_NGCHUNK = 2


def kernel(x, edge_index, edge_attr, W_gcn, b_gcn, Ws):
    ew = jnp.transpose(edge_attr[:, :, 2:6], (0, 2, 1))
    ew = ew.reshape(G, 4, _NS, _NCH, _CH)
    gs_parts, xs_parts = [], []
    for g0 in range(0, G, _NGCHUNK):
        Bc = _build_B_sc(edge_index, ew, 4 * g0, 4 * _NGCHUNK)
        gsc, xsc = _run_gcn(x, W_gcn, b_gcn, Ws, Bc, g0, _NGCHUNK)
        gs_parts.append(gsc)
        xs_parts.append(xsc)
    gs = jnp.concatenate(gs_parts, axis=0)
    xs = jnp.concatenate(xs_parts, axis=0)
    g_matrix = gs.reshape(1, G * OUT)
    sim = _run_sim(xs)
    return g_matrix, sim[None]


# speedup vs baseline: 1.0218x; 1.0218x over previous
import functools

import jax
import jax.numpy as jnp
from jax import lax
from jax.experimental import pallas as pl
from jax.experimental.pallas import tpu as pltpu
from jax.experimental.pallas import tpu_sc as plsc

G, N, E, D, OUT, DE = 6, 1024, 32768, 128, 128, 16
GI = G * 4

_INTERPRET = False


def _gcn_body(x_ref, w_ref, b_ref, ws_ref, Bt_ref,
              gs_ref, xs_ref, acc_ref):
    i = pl.program_id(1)
    xw = jnp.dot(x_ref[0], w_ref[0], preferred_element_type=jnp.float32)
    B2b = [Bt_ref[0, :, tc].reshape(N, OUT).astype(jnp.bfloat16)
           for tc in range(8)]
    ones_b = jnp.ones((OUT, 128), jnp.bfloat16)
    dcol = jnp.zeros((N, 128), jnp.float32)
    for tc in range(8):
        dcol = dcol + jnp.dot(B2b[tc], ones_b,
                              preferred_element_type=jnp.float32)
    deg = dcol[:, :1]
    dis = jax.lax.rsqrt(deg + 1.0)
    z = dis * xw
    zb = z.astype(jnp.bfloat16)
    m = jnp.zeros((N, OUT), jnp.float32)
    for tc in range(8):
        m = m + jnp.dot(B2b[tc], zb[tc * 128:(tc + 1) * 128, :],
                        preferred_element_type=jnp.float32)
    y = dis * m + (dis * dis) * xw + b_ref[0]
    xt = jnp.maximum(y, 0.0)
    contrib = jnp.dot(xt, ws_ref[0], preferred_element_type=jnp.float32)

    @pl.when(i == 0)
    def _():
        acc_ref[...] = contrib

    @pl.when(i != 0)
    def _():
        acc_ref[...] = acc_ref[...] + contrib

    @pl.when(i == 3)
    def _():
        t = acc_ref[...]
        gs_ref[0, 0] = jnp.mean(t, axis=0)
        nrm = jnp.sqrt(jnp.sum(t * t, axis=1, keepdims=True))
        xs_ref[0] = t / jnp.maximum(nrm, 1e-12)


def _run_gcn(x, W_gcn, b_gcn, Ws, Bt, g0, ng):
    b3 = b_gcn.reshape(4, 1, OUT)
    gs, xs = pl.pallas_call(
        _gcn_body,
        grid=(ng, 4),
        in_specs=[
            pl.BlockSpec((1, N, D), lambda g, i: (g0 + g, 0, 0)),
            pl.BlockSpec((1, D, OUT), lambda g, i: (i, 0, 0)),
            pl.BlockSpec((1, 1, OUT), lambda g, i: (i, 0, 0)),
            pl.BlockSpec((1, OUT, OUT), lambda g, i: (i, 0, 0)),
            pl.BlockSpec((1, 128, 8, 8, 128),
                         lambda g, i: (g * 4 + i, 0, 0, 0, 0)),
        ],
        out_specs=[
            pl.BlockSpec((1, 1, OUT), lambda g, i: (g, 0, 0)),
            pl.BlockSpec((1, N, OUT), lambda g, i: (g, 0, 0)),
        ],
        out_shape=[
            jax.ShapeDtypeStruct((ng, 1, OUT), jnp.float32),
            jax.ShapeDtypeStruct((ng, N, OUT), jnp.float32),
        ],
        scratch_shapes=[pltpu.VMEM((N, OUT), jnp.float32)],
        interpret=_INTERPRET,
    )(x, W_gcn, b3, Ws, Bt)
    return gs, xs


def _sim_body(a_ref, b_ref, out_ref):
    a16 = a_ref[0].astype(jnp.bfloat16)
    b16 = b_ref[0].astype(jnp.bfloat16)
    p = jax.lax.dot_general(a16, b16, (((1,), (1,)), ((), ())),
                            preferred_element_type=jnp.float32)
    t = jnp.tanh(p)
    f = jnp.floor((t + 1.0) * 10.5) - 10.0
    col = jnp.sum(f, axis=0)
    out_ref[0, 0] = jnp.sum(col.reshape(8, OUT), axis=0)


def _pair_i(p):
    return ((p >= 6).astype(jnp.int32) + (p >= 11) + (p >= 15)
            + (p >= 18) + (p >= 20))


def _pair_j(p):
    i = _pair_i(p)
    row_start = i * G - (i * (i - 1)) // 2
    return p - row_start + i


def _run_sim(xs):
    npairs = (G * (G + 1)) // 2
    part = pl.pallas_call(
        _sim_body,
        grid=(npairs,),
        in_specs=[
            pl.BlockSpec((1, N, OUT), lambda p: (_pair_i(p), 0, 0)),
            pl.BlockSpec((1, N, OUT), lambda p: (_pair_j(p), 0, 0)),
        ],
        out_specs=pl.BlockSpec((1, 1, OUT), lambda p: (p, 0, 0)),
        out_shape=jax.ShapeDtypeStruct((npairs, 1, OUT), jnp.float32),
        interpret=_INTERPRET,
    )(xs, xs)
    sums = jnp.sum(part.reshape(npairs, OUT), axis=-1)
    vals = (0.1 / (N * N)) * sums
    iu, ju = jnp.triu_indices(G)
    s = jnp.zeros((G, G), jnp.float32).at[iu, ju].set(vals)
    return jnp.where(jnp.triu(jnp.ones((G, G), jnp.bool_)), s, s.T)


_NS = 16
_L = 16
_EPT = E // _NS
_CH = 128
_NCH = _EPT // _CH
_STRIPE = (N * N) // _NS
_ZB = 16384


def _sc_build_body(ei_hbm, ew_hbm, B_hbm,
                   B_sh, zeros_v, dstv, srcv, idx2d, val2d, zval2d,
                   *, j0, nj):
    c = lax.axis_index("c")
    s = lax.axis_index("s")

    def _zinit(k, _):
        zeros_v[pl.ds(k * _L, _L)] = jnp.zeros((_L,), jnp.float32)
        return _
    lax.fori_loop(0, _ZB // _L, _zinit, None)

    def _zinit2(k, _):
        zval2d[k // (_CH // _L), pl.ds((k % (_CH // _L)) * _L, _L)] = \
            jnp.zeros((_L,), jnp.float32)
        return _
    lax.fori_loop(0, _EPT // _L, _zinit2, None)

    for q in range(_STRIPE // _ZB):
        pltpu.sync_copy(zeros_v, B_sh.at[pl.ds(s * _STRIPE + q * _ZB, _ZB)])

    def _round(r, _):
        j = j0 + 2 * r + c
        g = j // 4
        i = j % 4

        pltpu.sync_copy(ei_hbm.at[g, 0, pl.ds(s * _EPT, _EPT)], srcv)
        pltpu.sync_copy(ei_hbm.at[g, 1, pl.ds(s * _EPT, _EPT)], dstv)
        pltpu.sync_copy(ew_hbm.at[g, i, s], val2d)

        for t in range(_NCH):
            def _grp(kk, _):
                k = t * (_CH // _L) + kk
                d16 = dstv[pl.ds(k * _L, _L)]
                s16 = srcv[pl.ds(k * _L, _L)]
                addr = ((d16 >> 3) << 13) | ((s16 >> 7) << 10) \
                    | ((d16 & 7) << 7) | (s16 & 127)
                idx2d[t, pl.ds(kk * _L, _L)] = addr
                return _
            lax.fori_loop(0, _CH // _L, _grp, None)

        plsc.subcore_barrier()

        for t in range(_NCH):
            pltpu.sync_copy(val2d.at[t], B_sh.at[idx2d.at[t]], add=True)
        plsc.subcore_barrier()

        pltpu.sync_copy(B_sh.at[pl.ds(s * _STRIPE, _STRIPE)],
                        B_hbm.at[pl.ds((j - j0) * (N * N) + s * _STRIPE,
                                       _STRIPE)])
        plsc.subcore_barrier()

        @pl.when(r < nj // 2 - 1)
        def _():
            for t in range(_NCH):
                pltpu.sync_copy(zval2d.at[t], B_sh.at[idx2d.at[t]])
        return _

    lax.fori_loop(0, nj // 2, _round, None)


def _build_B_sc(edge_index, ew, j0, nj):
    run = pl.kernel(
        functools.partial(_sc_build_body, j0=j0, nj=nj),
        mesh=plsc.VectorSubcoreMesh(core_axis_name="c", subcore_axis_name="s"),
        out_type=jax.ShapeDtypeStruct((nj * N * N,), jnp.float32),
        scratch_types=[
            pltpu.VMEM_SHARED((N * N,), jnp.float32),
            pltpu.VMEM((_ZB,), jnp.float32),
            pltpu.VMEM((_EPT,), jnp.int32),
            pltpu.VMEM((_EPT,), jnp.int32),
            pltpu.VMEM((_NCH, _CH), jnp.int32),
            pltpu.VMEM((_NCH, _CH), jnp.float32),
            pltpu.VMEM((_NCH, _CH), jnp.float32),
        ],
    )
    B = run(edge_index, ew)
    return B.reshape(nj, N // 8, 8, 8, 128)


_NGCHUNK = 3


def kernel(x, edge_index, edge_attr, W_gcn, b_gcn, Ws):
    ew = jnp.transpose(edge_attr[:, :, 2:6], (0, 2, 1))
    ew = ew.reshape(G, 4, _NS, _NCH, _CH)
    gs_parts, xs_parts = [], []
    for g0 in range(0, G, _NGCHUNK):
        Bc = _build_B_sc(edge_index, ew, 4 * g0, 4 * _NGCHUNK)
        gsc, xsc = _run_gcn(x, W_gcn, b_gcn, Ws, Bc, g0, _NGCHUNK)
        gs_parts.append(gsc)
        xs_parts.append(xsc)
    gs = jnp.concatenate(gs_parts, axis=0)
    xs = jnp.concatenate(xs_parts, axis=0)
    g_matrix = gs.reshape(1, G * OUT)
    sim = _run_sim(xs)
    return g_matrix, sim[None]


# speedup vs baseline: 1.1073x; 1.0837x over previous
import functools

import jax
import jax.numpy as jnp
from jax import lax
from jax.experimental import pallas as pl
from jax.experimental.pallas import tpu as pltpu
from jax.experimental.pallas import tpu_sc as plsc

G, N, E, D, OUT, DE = 6, 1024, 32768, 128, 128, 16
GI = G * 4

_INTERPRET = False


def _gcn_body(x_ref, w_ref, b_ref, ws_ref, Bt_ref,
              gs_ref, xs_ref, acc_ref):
    i = pl.program_id(1)
    xw = jnp.dot(x_ref[0], w_ref[0], preferred_element_type=jnp.float32)
    B2b = [Bt_ref[0, :, tc].reshape(N, OUT).astype(jnp.bfloat16)
           for tc in range(8)]
    ones_b = jnp.ones((OUT, 128), jnp.bfloat16)
    dcol = jnp.zeros((N, 128), jnp.float32)
    for tc in range(8):
        dcol = dcol + jnp.dot(B2b[tc], ones_b,
                              preferred_element_type=jnp.float32)
    deg = dcol[:, :1]
    dis = jax.lax.rsqrt(deg + 1.0)
    z = dis * xw
    zb = z.astype(jnp.bfloat16)
    m = jnp.zeros((N, OUT), jnp.float32)
    for tc in range(8):
        m = m + jnp.dot(B2b[tc], zb[tc * 128:(tc + 1) * 128, :],
                        preferred_element_type=jnp.float32)
    y = dis * m + (dis * dis) * xw + b_ref[0]
    xt = jnp.maximum(y, 0.0)
    contrib = jnp.dot(xt, ws_ref[0], preferred_element_type=jnp.float32)

    @pl.when(i == 0)
    def _():
        acc_ref[...] = contrib

    @pl.when(i != 0)
    def _():
        acc_ref[...] = acc_ref[...] + contrib

    @pl.when(i == 3)
    def _():
        t = acc_ref[...]
        gs_ref[0, 0] = jnp.mean(t, axis=0)
        nrm = jnp.sqrt(jnp.sum(t * t, axis=1, keepdims=True))
        xs_ref[0] = t / jnp.maximum(nrm, 1e-12)


def _run_gcn(x, W_gcn, b_gcn, Ws, Bt, g0, ng):
    b3 = b_gcn.reshape(4, 1, OUT)
    gs, xs = pl.pallas_call(
        _gcn_body,
        grid=(ng, 4),
        in_specs=[
            pl.BlockSpec((1, N, D), lambda g, i: (g0 + g, 0, 0)),
            pl.BlockSpec((1, D, OUT), lambda g, i: (i, 0, 0)),
            pl.BlockSpec((1, 1, OUT), lambda g, i: (i, 0, 0)),
            pl.BlockSpec((1, OUT, OUT), lambda g, i: (i, 0, 0)),
            pl.BlockSpec((1, 128, 8, 8, 128),
                         lambda g, i: (g * 4 + i, 0, 0, 0, 0)),
        ],
        out_specs=[
            pl.BlockSpec((1, 1, OUT), lambda g, i: (g, 0, 0)),
            pl.BlockSpec((1, N, OUT), lambda g, i: (g, 0, 0)),
        ],
        out_shape=[
            jax.ShapeDtypeStruct((ng, 1, OUT), jnp.float32),
            jax.ShapeDtypeStruct((ng, N, OUT), jnp.float32),
        ],
        scratch_shapes=[pltpu.VMEM((N, OUT), jnp.float32)],
        interpret=_INTERPRET,
    )(x, W_gcn, b3, Ws, Bt)
    return gs, xs


def _sim_body(a_ref, b_ref, out_ref):
    a16 = a_ref[0].astype(jnp.bfloat16)
    b16 = b_ref[0].astype(jnp.bfloat16)
    p = jax.lax.dot_general(a16, b16, (((1,), (1,)), ((), ())),
                            preferred_element_type=jnp.float32)
    t = jnp.tanh(p)
    f = jnp.floor((t + 1.0) * 10.5) - 10.0
    col = jnp.sum(f, axis=0)
    out_ref[0, 0] = jnp.sum(col.reshape(8, OUT), axis=0)


def _pair_i(p):
    return ((p >= 6).astype(jnp.int32) + (p >= 11) + (p >= 15)
            + (p >= 18) + (p >= 20))


def _pair_j(p):
    i = _pair_i(p)
    row_start = i * G - (i * (i - 1)) // 2
    return p - row_start + i


def _run_sim(xs):
    npairs = (G * (G + 1)) // 2
    part = pl.pallas_call(
        _sim_body,
        grid=(npairs,),
        in_specs=[
            pl.BlockSpec((1, N, OUT), lambda p: (_pair_i(p), 0, 0)),
            pl.BlockSpec((1, N, OUT), lambda p: (_pair_j(p), 0, 0)),
        ],
        out_specs=pl.BlockSpec((1, 1, OUT), lambda p: (p, 0, 0)),
        out_shape=jax.ShapeDtypeStruct((npairs, 1, OUT), jnp.float32),
        interpret=_INTERPRET,
    )(xs, xs)
    sums = jnp.sum(part.reshape(npairs, OUT), axis=-1)
    vals = (0.1 / (N * N)) * sums
    iu, ju = jnp.triu_indices(G)
    s = jnp.zeros((G, G), jnp.float32).at[iu, ju].set(vals)
    return jnp.where(jnp.triu(jnp.ones((G, G), jnp.bool_)), s, s.T)


_NS = 16
_L = 16
_EPT = E // _NS
_CH = 128
_NCH = _EPT // _CH
_STRIPE = (N * N) // _NS
_ZB = 16384


def _sc_build_body(ei_hbm, ew_hbm, B_hbm,
                   B_sh, zeros_v, dstv, srcv, idx2d, val2d, zval2d, sem,
                   *, j0, nj):
    c = lax.axis_index("c")
    s = lax.axis_index("s")
    nrounds = nj // 2

    def _zinit(k, _):
        zeros_v[pl.ds(k * _L, _L)] = jnp.zeros((_L,), jnp.float32)
        return _
    lax.fori_loop(0, _ZB // _L, _zinit, None)

    def _zinit2(k, _):
        zval2d[k // (_CH // _L), pl.ds((k % (_CH // _L)) * _L, _L)] = \
            jnp.zeros((_L,), jnp.float32)
        return _
    lax.fori_loop(0, _EPT // _L, _zinit2, None)

    def _load_round(r, buf):
        j = j0 + 2 * r + c
        g = j // 4
        i = j % 4
        pltpu.sync_copy(ei_hbm.at[g, 0, pl.ds(s * _EPT, _EPT)], srcv)
        pltpu.sync_copy(ei_hbm.at[g, 1, pl.ds(s * _EPT, _EPT)], dstv)
        pltpu.sync_copy(ew_hbm.at[g, i, s], val2d.at[buf])
        for t in range(_NCH):
            def _grp(kk, _):
                k = t * (_CH // _L) + kk
                d16 = dstv[pl.ds(k * _L, _L)]
                s16 = srcv[pl.ds(k * _L, _L)]
                addr = ((d16 >> 3) << 13) | ((s16 >> 7) << 10) \
                    | ((d16 & 7) << 7) | (s16 & 127)
                idx2d[buf, t, pl.ds(kk * _L, _L)] = addr
                return _
            lax.fori_loop(0, _CH // _L, _grp, None)

    for q in range(_STRIPE // _ZB):
        pltpu.sync_copy(zeros_v, B_sh.at[pl.ds(s * _STRIPE + q * _ZB, _ZB)])
    _load_round(0, 0)

    def _round(r, _):
        j = j0 + 2 * r + c
        p = r % 2

        plsc.subcore_barrier()

        for t in range(_NCH):
            pltpu.sync_copy(val2d.at[p, t], B_sh.at[idx2d.at[p, t]], add=True)
        plsc.subcore_barrier()

        cp = pltpu.async_copy(
            B_sh.at[pl.ds(s * _STRIPE, _STRIPE)],
            B_hbm.at[pl.ds((j - j0) * (N * N) + s * _STRIPE, _STRIPE)], sem)

        @pl.when(r < nrounds - 1)
        def _():
            _load_round(r + 1, 1 - p)
        cp.wait()
        plsc.subcore_barrier()

        @pl.when(r < nrounds - 1)
        def _():
            for t in range(_NCH):
                pltpu.sync_copy(zval2d.at[t], B_sh.at[idx2d.at[p, t]])
        return _

    lax.fori_loop(0, nrounds, _round, None)


def _build_B_sc(edge_index, ew, j0, nj):
    run = pl.kernel(
        functools.partial(_sc_build_body, j0=j0, nj=nj),
        mesh=plsc.VectorSubcoreMesh(core_axis_name="c", subcore_axis_name="s"),
        out_type=jax.ShapeDtypeStruct((nj * N * N,), jnp.float32),
        scratch_types=[
            pltpu.VMEM_SHARED((N * N,), jnp.float32),
            pltpu.VMEM((_ZB,), jnp.float32),
            pltpu.VMEM((_EPT,), jnp.int32),
            pltpu.VMEM((_EPT,), jnp.int32),
            pltpu.VMEM((2, _NCH, _CH), jnp.int32),
            pltpu.VMEM((2, _NCH, _CH), jnp.float32),
            pltpu.VMEM((_NCH, _CH), jnp.float32),
            pltpu.SemaphoreType.DMA,
        ],
    )
    B = run(edge_index, ew)
    return B.reshape(nj, N // 8, 8, 8, 128)


_NGCHUNK = 3


def kernel(x, edge_index, edge_attr, W_gcn, b_gcn, Ws):
    ew = jnp.transpose(edge_attr[:, :, 2:6], (0, 2, 1))
    ew = ew.reshape(G, 4, _NS, _NCH, _CH)
    gs_parts, xs_parts = [], []
    for g0 in range(0, G, _NGCHUNK):
        Bc = _build_B_sc(edge_index, ew, 4 * g0, 4 * _NGCHUNK)
        gsc, xsc = _run_gcn(x, W_gcn, b_gcn, Ws, Bc, g0, _NGCHUNK)
        gs_parts.append(gsc)
        xs_parts.append(xsc)
    gs = jnp.concatenate(gs_parts, axis=0)
    xs = jnp.concatenate(xs_parts, axis=0)
    g_matrix = gs.reshape(1, G * OUT)
    sim = _run_sim(xs)
    return g_matrix, sim[None]


# speedup vs baseline: 1.1076x; 1.0003x over previous
import functools

import jax
import jax.numpy as jnp
from jax import lax
from jax.experimental import pallas as pl
from jax.experimental.pallas import tpu as pltpu
from jax.experimental.pallas import tpu_sc as plsc

G, N, E, D, OUT, DE = 6, 1024, 32768, 128, 128, 16
GI = G * 4

def _gcn_body(x_ref, w_ref, b_ref, ws_ref, Bt_ref,
              gs_ref, xs_ref, acc_ref):
    i = pl.program_id(1)
    xw = jnp.dot(x_ref[0], w_ref[0], preferred_element_type=jnp.float32)
    B2b = [Bt_ref[0, :, tc].reshape(N, OUT).astype(jnp.bfloat16)
           for tc in range(8)]
    ones_b = jnp.ones((OUT, 128), jnp.bfloat16)
    dcol = jnp.zeros((N, 128), jnp.float32)
    for tc in range(8):
        dcol = dcol + jnp.dot(B2b[tc], ones_b,
                              preferred_element_type=jnp.float32)
    deg = dcol[:, :1]
    dis = jax.lax.rsqrt(deg + 1.0)
    z = dis * xw
    zb = z.astype(jnp.bfloat16)
    m = jnp.zeros((N, OUT), jnp.float32)
    for tc in range(8):
        m = m + jnp.dot(B2b[tc], zb[tc * 128:(tc + 1) * 128, :],
                        preferred_element_type=jnp.float32)
    y = dis * m + (dis * dis) * xw + b_ref[0]
    xt = jnp.maximum(y, 0.0)
    contrib = jnp.dot(xt, ws_ref[0], preferred_element_type=jnp.float32)

    @pl.when(i == 0)
    def _():
        acc_ref[...] = contrib

    @pl.when(i != 0)
    def _():
        acc_ref[...] = acc_ref[...] + contrib

    @pl.when(i == 3)
    def _():
        t = acc_ref[...]
        gs_ref[0, 0] = jnp.mean(t, axis=0)
        nrm = jnp.sqrt(jnp.sum(t * t, axis=1, keepdims=True))
        xs_ref[0] = t / jnp.maximum(nrm, 1e-12)


def _run_gcn(x, W_gcn, b_gcn, Ws, Bt, g0, ng):
    b3 = b_gcn.reshape(4, 1, OUT)
    gs, xs = pl.pallas_call(
        _gcn_body,
        grid=(ng, 4),
        in_specs=[
            pl.BlockSpec((1, N, D), lambda g, i: (g0 + g, 0, 0)),
            pl.BlockSpec((1, D, OUT), lambda g, i: (i, 0, 0)),
            pl.BlockSpec((1, 1, OUT), lambda g, i: (i, 0, 0)),
            pl.BlockSpec((1, OUT, OUT), lambda g, i: (i, 0, 0)),
            pl.BlockSpec((1, 128, 8, 8, 128),
                         lambda g, i: (g * 4 + i, 0, 0, 0, 0)),
        ],
        out_specs=[
            pl.BlockSpec((1, 1, OUT), lambda g, i: (g, 0, 0)),
            pl.BlockSpec((1, N, OUT), lambda g, i: (g, 0, 0)),
        ],
        out_shape=[
            jax.ShapeDtypeStruct((ng, 1, OUT), jnp.float32),
            jax.ShapeDtypeStruct((ng, N, OUT), jnp.float32),
        ],
        scratch_shapes=[pltpu.VMEM((N, OUT), jnp.float32)],
    )(x, W_gcn, b3, Ws, Bt)
    return gs, xs


def _sim_body(a_ref, b_ref, out_ref):
    a16 = a_ref[0].astype(jnp.bfloat16)
    b16 = b_ref[0].astype(jnp.bfloat16)
    p = jax.lax.dot_general(a16, b16, (((1,), (1,)), ((), ())),
                            preferred_element_type=jnp.float32)
    t = jnp.tanh(p)
    f = jnp.floor((t + 1.0) * 10.5) - 10.0
    col = jnp.sum(f, axis=0)
    out_ref[0, 0] = jnp.sum(col.reshape(8, OUT), axis=0)


def _pair_i(p):
    return ((p >= 6).astype(jnp.int32) + (p >= 11) + (p >= 15)
            + (p >= 18) + (p >= 20))


def _pair_j(p):
    i = _pair_i(p)
    row_start = i * G - (i * (i - 1)) // 2
    return p - row_start + i


def _run_sim(xs):
    npairs = (G * (G + 1)) // 2
    part = pl.pallas_call(
        _sim_body,
        grid=(npairs,),
        in_specs=[
            pl.BlockSpec((1, N, OUT), lambda p: (_pair_i(p), 0, 0)),
            pl.BlockSpec((1, N, OUT), lambda p: (_pair_j(p), 0, 0)),
        ],
        out_specs=pl.BlockSpec((1, 1, OUT), lambda p: (p, 0, 0)),
        out_shape=jax.ShapeDtypeStruct((npairs, 1, OUT), jnp.float32),
    )(xs, xs)
    sums = jnp.sum(part.reshape(npairs, OUT), axis=-1)
    vals = (0.1 / (N * N)) * sums
    iu, ju = jnp.triu_indices(G)
    s = jnp.zeros((G, G), jnp.float32).at[iu, ju].set(vals)
    return jnp.where(jnp.triu(jnp.ones((G, G), jnp.bool_)), s, s.T)


_NS = 16
_L = 16
_EPT = E // _NS
_CH = 128
_NCH = _EPT // _CH
_STRIPE = (N * N) // _NS
_ZB = 16384


def _sc_build_body(ei_hbm, ew_hbm, B_hbm,
                   B_sh, zeros_v, dstv, srcv, idx2d, val2d, zval2d, sem,
                   *, j0, nj):
    c = lax.axis_index("c")
    s = lax.axis_index("s")
    nrounds = nj // 2

    def _zinit(k, _):
        zeros_v[pl.ds(k * _L, _L)] = jnp.zeros((_L,), jnp.float32)
        return _
    lax.fori_loop(0, _ZB // _L, _zinit, None)

    def _zinit2(k, _):
        zval2d[k // (_CH // _L), pl.ds((k % (_CH // _L)) * _L, _L)] = \
            jnp.zeros((_L,), jnp.float32)
        return _
    lax.fori_loop(0, _EPT // _L, _zinit2, None)

    def _load_round(r, buf):
        j = j0 + 2 * r + c
        g = j // 4
        i = j % 4
        pltpu.sync_copy(ei_hbm.at[g, 0, pl.ds(s * _EPT, _EPT)], srcv)
        pltpu.sync_copy(ei_hbm.at[g, 1, pl.ds(s * _EPT, _EPT)], dstv)
        pltpu.sync_copy(ew_hbm.at[g, i, s], val2d.at[buf])
        for t in range(_NCH):
            def _grp(kk, _):
                k = t * (_CH // _L) + kk
                d16 = dstv[pl.ds(k * _L, _L)]
                s16 = srcv[pl.ds(k * _L, _L)]
                addr = ((d16 >> 3) << 13) | ((s16 >> 7) << 10) \
                    | ((d16 & 7) << 7) | (s16 & 127)
                idx2d[buf, t, pl.ds(kk * _L, _L)] = addr
                return _
            lax.fori_loop(0, _CH // _L, _grp, None)

    for q in range(_STRIPE // _ZB):
        pltpu.sync_copy(zeros_v, B_sh.at[pl.ds(s * _STRIPE + q * _ZB, _ZB)])
    _load_round(0, 0)

    def _round(r, _):
        j = j0 + 2 * r + c
        p = r % 2

        plsc.subcore_barrier()

        for t in range(_NCH):
            pltpu.sync_copy(val2d.at[p, t], B_sh.at[idx2d.at[p, t]], add=True)
        plsc.subcore_barrier()

        cp = pltpu.async_copy(
            B_sh.at[pl.ds(s * _STRIPE, _STRIPE)],
            B_hbm.at[pl.ds((j - j0) * (N * N) + s * _STRIPE, _STRIPE)], sem)

        @pl.when(r < nrounds - 1)
        def _():
            _load_round(r + 1, 1 - p)
        cp.wait()
        plsc.subcore_barrier()

        @pl.when(r < nrounds - 1)
        def _():
            for t in range(_NCH):
                pltpu.sync_copy(zval2d.at[t], B_sh.at[idx2d.at[p, t]])
        return _

    lax.fori_loop(0, nrounds, _round, None)


def _build_B_sc(edge_index, ew, j0, nj):
    run = pl.kernel(
        functools.partial(_sc_build_body, j0=j0, nj=nj),
        mesh=plsc.VectorSubcoreMesh(core_axis_name="c", subcore_axis_name="s"),
        out_type=jax.ShapeDtypeStruct((nj * N * N,), jnp.float32),
        scratch_types=[
            pltpu.VMEM_SHARED((N * N,), jnp.float32),
            pltpu.VMEM((_ZB,), jnp.float32),
            pltpu.VMEM((_EPT,), jnp.int32),
            pltpu.VMEM((_EPT,), jnp.int32),
            pltpu.VMEM((2, _NCH, _CH), jnp.int32),
            pltpu.VMEM((2, _NCH, _CH), jnp.float32),
            pltpu.VMEM((_NCH, _CH), jnp.float32),
            pltpu.SemaphoreType.DMA,
        ],
    )
    B = run(edge_index, ew)
    return B.reshape(nj, N // 8, 8, 8, 128)


_NGCHUNK = 3


def kernel(x, edge_index, edge_attr, W_gcn, b_gcn, Ws):
    ew = jnp.transpose(edge_attr[:, :, 2:6], (0, 2, 1))
    ew = ew.reshape(G, 4, _NS, _NCH, _CH)
    gs_parts, xs_parts = [], []
    for g0 in range(0, G, _NGCHUNK):
        Bc = _build_B_sc(edge_index, ew, 4 * g0, 4 * _NGCHUNK)
        gsc, xsc = _run_gcn(x, W_gcn, b_gcn, Ws, Bc, g0, _NGCHUNK)
        gs_parts.append(gsc)
        xs_parts.append(xsc)
    gs = jnp.concatenate(gs_parts, axis=0)
    xs = jnp.concatenate(xs_parts, axis=0)
    g_matrix = gs.reshape(1, G * OUT)
    sim = _run_sim(xs)
    return g_matrix, sim[None]
